# baseline (speedup 1.0000x reference)
import functools

import jax
import jax.numpy as jnp
from jax import lax
from jax.experimental import pallas as pl
from jax.experimental.pallas import tpu as pltpu
from jax.experimental.pallas import tpu_sc as plsc

B = 16384
D = 64

_info = plsc.get_sparse_core_info()
_NC = _info.num_cores
_NS = _info.num_subcores
_L = _info.num_lanes
_NW = _NC * _NS
_RPW = B // _NW
_CHUNKS = D // _L

_mesh = plsc.VectorSubcoreMesh(core_axis_name="c", subcore_axis_name="s")


@functools.partial(
    pl.kernel,
    mesh=_mesh,
    out_type=jax.ShapeDtypeStruct((B,), jnp.float32),
    compiler_params=pltpu.CompilerParams(needs_layout_passes=False),
    scratch_types=[
        pltpu.VMEM((_RPW * D,), jnp.float32),
        pltpu.VMEM((_RPW * D,), jnp.float32),
        pltpu.VMEM((_RPW,), jnp.float32),
        pltpu.VMEM((_L * _L,), jnp.float32),
    ],
)
def _sc_rowdot(gu_hbm, gi_hbm, out_hbm, gu_v, gi_v, out_v, t_v):
    wid = lax.axis_index("s") * _NC + lax.axis_index("c")
    base = wid * _RPW
    pltpu.sync_copy(gu_hbm.at[pl.ds(base * D, _RPW * D)], gu_v)
    pltpu.sync_copy(gi_hbm.at[pl.ds(base * D, _RPW * D)], gi_v)

    rows = lax.iota(jnp.int32, 16) * _L

    def group(g, carry):
        for j in range(_L):
            off = g * (_L * D) + j * D
            p = gu_v[pl.ds(off, _L)] * gi_v[pl.ds(off, _L)]
            for k in range(1, _CHUNKS):
                p = p + (gu_v[pl.ds(off + k * _L, _L)]
                         * gi_v[pl.ds(off + k * _L, _L)])
            t_v[pl.ds(j * _L, _L)] = p
        res = plsc.load_gather(t_v, [rows])
        for l in range(1, _L):
            res = res + plsc.load_gather(t_v, [rows + l])
        out_v[pl.ds(g * _L, _L)] = res
        return carry

    lax.fori_loop(0, _RPW // _L, group, 0)
    pltpu.sync_copy(out_v, out_hbm.at[pl.ds(base, _RPW)])


def kernel(gu, gi):
    gu = jnp.squeeze(gu).reshape(B * D)
    gi = jnp.squeeze(gi).reshape(B * D)
    return _sc_rowdot(gu, gi)


# speedup vs baseline: 1.1431x; 1.1431x over previous
import functools

import jax
import jax.numpy as jnp
from jax import lax
from jax.experimental import pallas as pl
from jax.experimental.pallas import tpu as pltpu
from jax.experimental.pallas import tpu_sc as plsc

B = 16384
D = 64

_info = plsc.get_sparse_core_info()
_NC = _info.num_cores
_NS = _info.num_subcores
_L = _info.num_lanes
_NW = _NC * _NS
_RPW = B // _NW
_CHUNKS = D // _L
_NSTEP = 4
_RPC = _RPW // _NSTEP
_GPC = _RPC // _L

_mesh = plsc.VectorSubcoreMesh(core_axis_name="c", subcore_axis_name="s")


@functools.partial(
    pl.kernel,
    mesh=_mesh,
    out_type=jax.ShapeDtypeStruct((B,), jnp.float32),
    compiler_params=pltpu.CompilerParams(needs_layout_passes=False),
    scratch_types=[
        pltpu.VMEM((_RPW * D,), jnp.float32),
        pltpu.VMEM((_RPW * D,), jnp.float32),
        pltpu.VMEM((_RPW,), jnp.float32),
        pltpu.VMEM((_GPC * _L * _L,), jnp.float32),
        pltpu.SemaphoreType.DMA((_NSTEP,)),
    ],
)
def _sc_rowdot(gu_hbm, gi_hbm, out_hbm, gu_v, gi_v, out_v, t_v, sems):
    wid = lax.axis_index("s") * _NC + lax.axis_index("c")
    base = wid * _RPW

    copies = []
    for c in range(_NSTEP):
        off = (base + c * _RPC) * D
        loc = c * _RPC * D
        copies.append((
            pltpu.async_copy(
                gu_hbm.at[pl.ds(off, _RPC * D)],
                gu_v.at[pl.ds(loc, _RPC * D)], sems.at[c]),
            pltpu.async_copy(
                gi_hbm.at[pl.ds(off, _RPC * D)],
                gi_v.at[pl.ds(loc, _RPC * D)], sems.at[c]),
        ))

    rows = lax.iota(jnp.int32, 16) * _L

    for c in range(_NSTEP):
        for h in copies[c]:
            h.wait()

        @functools.partial(plsc.parallel_loop, 0, _GPC, unroll=2)
        def _chunk_body(g):
            tbase = g * (_L * _L)
            for j in range(_L):
                off = c * (_RPC * D) + g * (_L * D) + j * D
                p = gu_v[pl.ds(off, _L)] * gi_v[pl.ds(off, _L)]
                for k in range(1, _CHUNKS):
                    p = p + (gu_v[pl.ds(off + k * _L, _L)]
                             * gi_v[pl.ds(off + k * _L, _L)])
                t_v[pl.ds(tbase + j * _L, _L)] = p
            res = plsc.load_gather(t_v, [tbase + rows])
            for l in range(1, _L):
                res = res + plsc.load_gather(t_v, [tbase + rows + l])
            out_v[pl.ds(c * _RPC + g * _L, _L)] = res

    pltpu.sync_copy(out_v, out_hbm.at[pl.ds(base, _RPW)])


def kernel(gu, gi):
    gu = jnp.squeeze(gu).reshape(B * D)
    gi = jnp.squeeze(gi).reshape(B * D)
    return _sc_rowdot(gu, gi)
